# baseline (speedup 1.0000x reference)
import functools

import jax
import jax.numpy as jnp
from jax import lax
from jax.experimental import pallas as pl
from jax.experimental.pallas import tpu as pltpu
from jax.experimental.pallas import tpu_sc as plsc

N_R = 10000
N_N = 10000
E = 160000
D = 128
L = 4

NC = 2
NS = 16
NW = NC * NS
CHUNK = 128
NPAD = 10240
EPAD = 163840
CPW = EPAD // CHUNK // NW
RPW = NPAD // NS
PADV = NPAD - 1


NBUF = 2


def _sc_agg_body(table, src, dst, out, idx_s, idx_d, rows, acc, sem_g,
                 sem_s):
    c = lax.axis_index("c")
    s = lax.axis_index("s")
    w = s * NC + c

    pltpu.sync_copy(src.at[w], idx_s)
    pltpu.sync_copy(dst.at[w], idx_d)
    for b in range(1, NBUF):
        pltpu.async_copy(table.at[idx_s.at[b]], rows.at[b], sem_g.at[b])

    zv = jnp.zeros((16,), jnp.float32)

    def zstep(i, _):
        r = i // (D // 16)
        col = (i % (D // 16)) * 16
        rows[0, r, pl.ds(col, 16)] = zv
        return 0

    lax.fori_loop(0, CHUNK * (D // 16), zstep, 0)
    for k in range(RPW // CHUNK):
        pltpu.sync_copy(rows.at[0], acc.at[pl.ds(s * RPW + k * CHUNK, CHUNK)])
    pltpu.async_copy(table.at[idx_s.at[0]], rows.at[0], sem_g.at[0])
    plsc.subcore_barrier()

    def group(g, _):
        for b in range(NBUF):
            j = g * NBUF + b
            pltpu.make_async_copy(table.at[idx_s.at[b]], rows.at[b],
                                  sem_g.at[b]).wait()
            pltpu.async_copy(rows.at[b], acc.at[idx_d.at[j]], sem_s.at[b],
                             add=True).wait()

            @pl.when(j + NBUF < CPW)
            def _():
                pltpu.async_copy(table.at[idx_s.at[j + NBUF]], rows.at[b],
                                 sem_g.at[b])
        return 0

    lax.fori_loop(0, CPW // NBUF, group, 0)
    plsc.subcore_barrier()

    pltpu.sync_copy(acc.at[pl.ds(s * RPW, RPW)],
                    out.at[c].at[pl.ds(s * RPW, RPW)])


N_TAB = N_N
def _unpack_src(idx_p, sidx, b, j, base):
    for k in range(CHUNK // 16):
        v = idx_p[j, pl.ds(k * 16, 16)]
        sidx[b, pl.ds(k * 16, 16)] = jnp.bitwise_and(v, 0xFFFF) + base


def _unpack_dst(idx_p, didx, b, j):
    for k in range(CHUNK // 16):
        v = idx_p[j, pl.ds(k * 16, 16)]
        didx[b, pl.ds(k * 16, 16)] = lax.shift_right_logical(v, 16)


def _sc_agg2_body(tab2, idxp, out, idx_p, sidx, didx, rows, acc, sem_g,
                  sem_s):
    c = lax.axis_index("c")
    s = lax.axis_index("s")

    pltpu.sync_copy(idxp.at[c].at[s], idx_p)
    base = c * N_TAB
    for b in range(1, NBUF):
        _unpack_src(idx_p, sidx, b, b, base)
        pltpu.async_copy(tab2.at[sidx.at[b]], rows.at[b], sem_g.at[b])

    zv = jnp.zeros((16,), jnp.float32)

    def zstep(i, _):
        r = i // (D // 16)
        col = (i % (D // 16)) * 16
        rows[0, r, pl.ds(col, 16)] = zv
        return 0

    lax.fori_loop(0, CHUNK * (D // 16), zstep, 0)
    for k in range(RPW // CHUNK):
        pltpu.sync_copy(rows.at[0], acc.at[pl.ds(s * RPW + k * CHUNK, CHUNK)])
    _unpack_src(idx_p, sidx, 0, 0, base)
    pltpu.async_copy(tab2.at[sidx.at[0]], rows.at[0], sem_g.at[0])
    plsc.subcore_barrier()

    def group(g, _):
        for b in range(NBUF):
            j = g * NBUF + b
            pltpu.make_async_copy(tab2.at[sidx.at[b]], rows.at[b],
                                  sem_g.at[b]).wait()
            _unpack_dst(idx_p, didx, b, j)
            pltpu.async_copy(rows.at[b], acc.at[didx.at[b]], sem_s.at[b],
                             add=True).wait()

            @pl.when(j + NBUF < CPW2)
            def _():
                _unpack_src(idx_p, sidx, b, j + NBUF, base)
                pltpu.async_copy(tab2.at[sidx.at[b]], rows.at[b], sem_g.at[b])
        return 0

    lax.fori_loop(0, CPW2 // NBUF, group, 0)
    plsc.subcore_barrier()

    pltpu.sync_copy(acc.at[pl.ds(s * RPW, RPW)],
                    out.at[c].at[pl.ds(s * RPW, RPW)])


def _sc_agg2(tab2, idxp):
    kern = pl.kernel(
        _sc_agg2_body,
        out_type=jax.ShapeDtypeStruct((NC, NPAD, D), jnp.float32),
        mesh=plsc.VectorSubcoreMesh(core_axis_name="c", subcore_axis_name="s",
                                    num_cores=NC, num_subcores=NS),
        scratch_types=[
            pltpu.VMEM((CPW2, CHUNK), jnp.int32),
            pltpu.VMEM((NBUF, CHUNK), jnp.int32),
            pltpu.VMEM((NBUF, CHUNK), jnp.int32),
            pltpu.VMEM((NBUF, CHUNK, D), jnp.float32),
            pltpu.VMEM_SHARED((NPAD, D), jnp.float32),
            pltpu.SemaphoreType.DMA((NBUF,)),
            pltpu.SemaphoreType.DMA((NBUF,)),
        ],
    )
    return kern(tab2, idxp)


CPW2 = EPAD // CHUNK // NS
CW = 128


def _sc_count_body(dsts, out, idx_d, ones_b, acc, sem_s):
    c = lax.axis_index("c")
    s = lax.axis_index("s")

    zv = jnp.zeros((16,), jnp.float32)
    ov = jnp.ones((16,), jnp.float32)

    def zstep(i, _):
        r = i // (CW // 16)
        col = (i % (CW // 16)) * 16
        ones_b[r, pl.ds(col, 16)] = zv
        return 0

    lax.fori_loop(0, CHUNK * (CW // 16), zstep, 0)
    for k in range(RPW // CHUNK):
        pltpu.sync_copy(ones_b, acc.at[pl.ds(s * RPW + k * CHUNK, CHUNK)])

    def ostep(i, _):
        r = i // (CW // 16)
        col = (i % (CW // 16)) * 16
        ones_b[r, pl.ds(col, 16)] = ov
        return 0

    lax.fori_loop(0, CHUNK * (CW // 16), ostep, 0)
    plsc.subcore_barrier()

    pltpu.sync_copy(dsts.at[c].at[s], idx_d)

    def step(j, _):
        pltpu.async_copy(ones_b, acc.at[idx_d.at[j]], sem_s, add=True).wait()
        return 0

    lax.fori_loop(0, CPW2, step, 0)
    plsc.subcore_barrier()

    pltpu.sync_copy(acc.at[pl.ds(s * RPW, RPW)],
                    out.at[c].at[pl.ds(s * RPW, RPW)])


def _sc_count(dsts):
    kern = pl.kernel(
        _sc_count_body,
        out_type=jax.ShapeDtypeStruct((NC, NPAD, CW), jnp.float32),
        mesh=plsc.VectorSubcoreMesh(core_axis_name="c", subcore_axis_name="s",
                                    num_cores=NC, num_subcores=NS),
        scratch_types=[
            pltpu.VMEM((CPW2, CHUNK), jnp.int32),
            pltpu.VMEM((CHUNK, CW), jnp.float32),
            pltpu.VMEM_SHARED((NPAD, CW), jnp.float32),
            pltpu.SemaphoreType.DMA,
        ],
    )
    return kern(dsts)


def _sc_agg(table, src, dst):
    kern = pl.kernel(
        _sc_agg_body,
        out_type=jax.ShapeDtypeStruct((NC, NPAD, D), jnp.float32),
        mesh=plsc.VectorSubcoreMesh(core_axis_name="c", subcore_axis_name="s",
                                    num_cores=NC, num_subcores=NS),
        scratch_types=[
            pltpu.VMEM((CPW, CHUNK), jnp.int32),
            pltpu.VMEM((CPW, CHUNK), jnp.int32),
            pltpu.VMEM((NBUF, CHUNK, D), jnp.float32),
            pltpu.VMEM_SHARED((NPAD, D), jnp.float32),
            pltpu.SemaphoreType.DMA((NBUF,)),
            pltpu.SemaphoreType.DMA((NBUF,)),
        ],
    )
    return kern(table, src, dst)


_PREC = lax.Precision.HIGHEST


def _mm(x, w):
    return lax.dot_general(x, w, (((1,), (1,)), ((), ())),
                           precision=_PREC, preferred_element_type=jnp.float32)


def _prologue_body(xn_ref, xr_ref, wl0_ref, wl1_ref, yn_ref, yr_ref):
    yn_ref[...] = _mm(xn_ref[...], wl0_ref[...])
    yr_ref[...] = _mm(xr_ref[...], wl1_ref[...])


def _prologue(xn, xr, wl0, wl1, blk=1000):
    grid = (N_N // blk,)
    return pl.pallas_call(
        _prologue_body,
        grid=grid,
        in_specs=[
            pl.BlockSpec((blk, D), lambda i: (i, 0)),
            pl.BlockSpec((blk, D), lambda i: (i, 0)),
            pl.BlockSpec((D, D), lambda i: (0, 0)),
            pl.BlockSpec((D, D), lambda i: (0, 0)),
        ],
        out_specs=[
            pl.BlockSpec((blk, D), lambda i: (i, 0)),
            pl.BlockSpec((blk, D), lambda i: (i, 0)),
        ],
        out_shape=[
            jax.ShapeDtypeStruct((N_N, D), jnp.float32),
            jax.ShapeDtypeStruct((N_R, D), jnp.float32),
        ],
    )(xn, xr, wl0, wl1)


def _mean_from_partials(sp_ref, cp_ref):
    ssum = sp_ref[0] + sp_ref[1]
    cnt = cp_ref[0][:, 0:1]
    return ssum / jnp.maximum(cnt, 1.0)


def _mean_merged(sp_ref, cp_ref):
    cnt = cp_ref[0][:, 0:1]
    return sp_ref[0] / jnp.maximum(cnt, 1.0)


def _combine_body(spr_ref, cpr_ref, spn_ref, cpn_ref, xr_ref, xn_ref,
                  wr0_ref, wr1_ref, b0_ref, b1_ref, wln0_ref, wln1_ref,
                  xr2_ref, xn2_ref, yn2_ref, yr2_ref):
    mean_r = _mean_merged(spr_ref, cpr_ref)
    mean_n = _mean_merged(spn_ref, cpn_ref)
    xr2 = mean_r + b0_ref[...] + _mm(xr_ref[...], wr0_ref[...])
    xn2 = mean_n + b1_ref[...] + _mm(xn_ref[...], wr1_ref[...])
    xr2_ref[...] = xr2
    xn2_ref[...] = xn2
    yn2_ref[...] = _mm(xn2, wln0_ref[...])
    yr2_ref[...] = _mm(xr2, wln1_ref[...])


def _cnt_spec(rel, blk):
    return pl.BlockSpec((1, blk, CW), lambda i, r=rel: (r, i, 0))


def _combine(sp2, cnt, xr, xn, wr0, wr1, b0, b1, wln0, wln1,
             blk=1000):
    grid = (N_R // blk,)
    rel0 = pl.BlockSpec((1, blk, D), lambda i: (0, i, 0))
    rel1 = pl.BlockSpec((1, blk, D), lambda i: (1, i, 0))
    rows = pl.BlockSpec((blk, D), lambda i: (i, 0))
    wspec = pl.BlockSpec((D, D), lambda i: (0, 0))
    bspec = pl.BlockSpec((1, D), lambda i: (0, 0))
    return pl.pallas_call(
        _combine_body,
        grid=grid,
        in_specs=[rel0, _cnt_spec(0, blk), rel1, _cnt_spec(1, blk),
                  rows, rows, wspec, wspec, bspec, bspec, wspec, wspec],
        out_specs=[rows, rows, rows, rows],
        out_shape=[jax.ShapeDtypeStruct((N_R, D), jnp.float32)] * 4,
    )(sp2, cnt, sp2, cnt, xr, xn, wr0, wr1, b0, b1, wln0, wln1)


def _final_body(spr_ref, cpr_ref, xr_ref, wr0_ref, b0_ref,
                wre_ref, bre_ref, notes_ref, wno_ref, bno_ref, out_ref):
    mean_r = _mean_from_partials(spr_ref, cpr_ref)
    xr2 = mean_r + b0_ref[...] + _mm(xr_ref[...], wr0_ref[...])
    a = _mm(xr2, wre_ref[...]) + bre_ref[...]
    b = _mm(notes_ref[...], wno_ref[...]) + bno_ref[...]
    eps = 1e-8
    num = jnp.sum(a * b, axis=1)
    na = jnp.maximum(jnp.sqrt(jnp.sum(a * a, axis=1)), eps)
    nb = jnp.maximum(jnp.sqrt(jnp.sum(b * b, axis=1)), eps)
    out_ref[...] = ((num / (na * nb) + 1.0) * 0.5).reshape(out_ref.shape)


def _final(spr, cnt, xr, wr0, b0, wre, bre, notes, wno, bno, blk=1000):
    grid = (N_R // blk,)
    part = pl.BlockSpec((NC, blk, D), lambda i: (0, i, 0))
    rows = pl.BlockSpec((blk, D), lambda i: (i, 0))
    wspec = pl.BlockSpec((D, D), lambda i: (0, 0))
    bspec = pl.BlockSpec((1, D), lambda i: (0, 0))
    return pl.pallas_call(
        _final_body,
        grid=grid,
        in_specs=[part, _cnt_spec(0, blk), rows, wspec, bspec,
                  wspec, bspec, rows, wspec, bspec],
        out_specs=pl.BlockSpec((1, 1, blk), lambda i: (i, 0, 0)),
        out_shape=jax.ShapeDtypeStruct((N_R // blk, 1, blk), jnp.float32),
    )(spr, cnt, xr, wr0, b0, wre, bre, notes, wno, bno)


def _prep_edges(edge_index):
    src = edge_index[0].astype(jnp.int32)
    dst = edge_index[1].astype(jnp.int32)
    pad = EPAD - E
    src = jnp.concatenate([src, jnp.zeros((pad,), jnp.int32)])
    dst = jnp.concatenate([dst, jnp.full((pad,), PADV, jnp.int32)])
    return src.reshape(NW, CPW, CHUNK), dst.reshape(NW, CPW, CHUNK)


def kernel(x_reaction, x_node, output_notes_opt, edge_index_n2r,
           edge_index_r2n, Wl, bl, Wr, W_reaction, b_reaction, W_nodes,
           b_nodes):
    src_nr, dst_nr = _prep_edges(edge_index_n2r)
    src_rn, dst_rn = _prep_edges(edge_index_r2n)

    dsts = jnp.stack([dst_nr.reshape(NS, CPW2, CHUNK),
                      dst_rn.reshape(NS, CPW2, CHUNK)])
    cnt = _sc_count(dsts)

    idxp = jnp.stack(
        [(src_nr + (dst_nr << 16)).reshape(NS, CPW2, CHUNK),
         (src_rn + (dst_rn << 16)).reshape(NS, CPW2, CHUNK)])

    xr, xn = x_reaction, x_node
    yn, yr = _prologue(xn, xr, Wl[0, 0], Wl[0, 1])
    for i in range(L - 1):
        sp2 = _sc_agg2(jnp.concatenate([yn, yr]), idxp)
        xr, xn, yn, yr = _combine(
            sp2, cnt, xr, xn,
            Wr[i, 0], Wr[i, 1],
            bl[i, 0].reshape(1, D), bl[i, 1].reshape(1, D),
            Wl[i + 1, 0], Wl[i + 1, 1])
    spr = _sc_agg(yn, src_nr, dst_nr)
    out = _final(spr, cnt, xr, Wr[L - 1, 0], bl[L - 1, 0].reshape(1, D),
                 W_reaction, b_reaction.reshape(1, D),
                 output_notes_opt, W_nodes, b_nodes.reshape(1, D))
    return out.reshape(N_R)


# speedup vs baseline: 1.0122x; 1.0122x over previous
import functools

import jax
import jax.numpy as jnp
from jax import lax
from jax.experimental import pallas as pl
from jax.experimental.pallas import tpu as pltpu
from jax.experimental.pallas import tpu_sc as plsc

N_R = 10000
N_N = 10000
E = 160000
D = 128
L = 4

NC = 2
NS = 16
NW = NC * NS
CHUNK = 128
NPAD = 10240
EPAD = 163840
CPW = EPAD // CHUNK // NW
RPW = NPAD // NS
PADV = NPAD - 1


NBUF = 2


def _sc_agg_body(table, src, dst, out, idx_s, idx_d, rows, acc, sem_g,
                 sem_s):
    c = lax.axis_index("c")
    s = lax.axis_index("s")
    w = s * NC + c

    pltpu.sync_copy(src.at[w], idx_s)
    pltpu.sync_copy(dst.at[w], idx_d)
    for b in range(1, NBUF):
        pltpu.async_copy(table.at[idx_s.at[b]], rows.at[b], sem_g.at[b])

    zv = jnp.zeros((16,), jnp.float32)

    def zstep(i, _):
        r = i // (D // 16)
        col = (i % (D // 16)) * 16
        rows[0, r, pl.ds(col, 16)] = zv
        return 0

    lax.fori_loop(0, CHUNK * (D // 16), zstep, 0)
    for k in range(RPW // CHUNK):
        pltpu.sync_copy(rows.at[0], acc.at[pl.ds(s * RPW + k * CHUNK, CHUNK)])
    pltpu.async_copy(table.at[idx_s.at[0]], rows.at[0], sem_g.at[0])
    plsc.subcore_barrier()

    def group(g, _):
        for b in range(NBUF):
            j = g * NBUF + b
            pltpu.make_async_copy(table.at[idx_s.at[b]], rows.at[b],
                                  sem_g.at[b]).wait()
            pltpu.async_copy(rows.at[b], acc.at[idx_d.at[j]], sem_s.at[b],
                             add=True).wait()

            @pl.when(j + NBUF < CPW)
            def _():
                pltpu.async_copy(table.at[idx_s.at[j + NBUF]], rows.at[b],
                                 sem_g.at[b])
        return 0

    lax.fori_loop(0, CPW // NBUF, group, 0)
    plsc.subcore_barrier()

    pltpu.sync_copy(acc.at[pl.ds(s * RPW, RPW)],
                    out.at[c].at[pl.ds(s * RPW, RPW)])


N_TAB = N_N
def _unpack_src(idx_p, sidx, b, j, base):
    for k in range(CHUNK // 16):
        v = idx_p[j, pl.ds(k * 16, 16)]
        sidx[b, pl.ds(k * 16, 16)] = jnp.bitwise_and(v, 0xFFFF) + base


def _unpack_dst(idx_p, didx, b, j):
    for k in range(CHUNK // 16):
        v = idx_p[j, pl.ds(k * 16, 16)]
        didx[b, pl.ds(k * 16, 16)] = lax.shift_right_logical(v, 16)


def _sc_agg2_body(tab2, idxp, out, idx_p, sidx, didx, rows, acc, sem_g,
                  sem_s):
    c = lax.axis_index("c")
    s = lax.axis_index("s")

    pltpu.sync_copy(idxp.at[c].at[s], idx_p)
    base = c * N_TAB
    for b in range(1, NBUF):
        _unpack_src(idx_p, sidx, b, b, base)
        pltpu.async_copy(tab2.at[sidx.at[b]], rows.at[b], sem_g.at[b])

    zv = jnp.zeros((16,), jnp.float32)

    def zstep(i, _):
        r = i // (D // 16)
        col = (i % (D // 16)) * 16
        rows[0, r, pl.ds(col, 16)] = zv
        return 0

    lax.fori_loop(0, CHUNK * (D // 16), zstep, 0)
    for k in range(RPW // CHUNK):
        pltpu.sync_copy(rows.at[0], acc.at[pl.ds(s * RPW + k * CHUNK, CHUNK)])
    _unpack_src(idx_p, sidx, 0, 0, base)
    pltpu.async_copy(tab2.at[sidx.at[0]], rows.at[0], sem_g.at[0])
    plsc.subcore_barrier()

    def group(g, _):
        for b in range(NBUF):
            j = g * NBUF + b
            pltpu.make_async_copy(tab2.at[sidx.at[b]], rows.at[b],
                                  sem_g.at[b]).wait()
            _unpack_dst(idx_p, didx, b, j)
            pltpu.async_copy(rows.at[b], acc.at[didx.at[b]], sem_s.at[b],
                             add=True).wait()

            @pl.when(j + NBUF < CPW2)
            def _():
                _unpack_src(idx_p, sidx, b, j + NBUF, base)
                pltpu.async_copy(tab2.at[sidx.at[b]], rows.at[b], sem_g.at[b])
        return 0

    lax.fori_loop(0, CPW2 // NBUF, group, 0)
    plsc.subcore_barrier()

    pltpu.sync_copy(acc.at[pl.ds(s * RPW, RPW)],
                    out.at[c].at[pl.ds(s * RPW, RPW)])


def _sc_agg2(tab2, idxp):
    kern = pl.kernel(
        _sc_agg2_body,
        out_type=jax.ShapeDtypeStruct((NC, NPAD, D), jnp.float32),
        mesh=plsc.VectorSubcoreMesh(core_axis_name="c", subcore_axis_name="s",
                                    num_cores=NC, num_subcores=NS),
        scratch_types=[
            pltpu.VMEM((CPW2, CHUNK), jnp.int32),
            pltpu.VMEM((NBUF, CHUNK), jnp.int32),
            pltpu.VMEM((NBUF, CHUNK), jnp.int32),
            pltpu.VMEM((NBUF, CHUNK, D), jnp.float32),
            pltpu.VMEM_SHARED((NPAD, D), jnp.float32),
            pltpu.SemaphoreType.DMA((NBUF,)),
            pltpu.SemaphoreType.DMA((NBUF,)),
        ],
    )
    return kern(tab2, idxp)


CPW2 = EPAD // CHUNK // NS
CW = 128


def _sc_count_body(dsts, out, idx_d, ones_b, acc, sem_s):
    c = lax.axis_index("c")
    s = lax.axis_index("s")

    zv = jnp.zeros((16,), jnp.float32)
    ov = jnp.ones((16,), jnp.float32)

    def zstep(i, _):
        r = i // (CW // 16)
        col = (i % (CW // 16)) * 16
        ones_b[r, pl.ds(col, 16)] = zv
        return 0

    lax.fori_loop(0, CHUNK * (CW // 16), zstep, 0)
    for k in range(RPW // CHUNK):
        pltpu.sync_copy(ones_b, acc.at[pl.ds(s * RPW + k * CHUNK, CHUNK)])

    def ostep(i, _):
        r = i // (CW // 16)
        col = (i % (CW // 16)) * 16
        ones_b[r, pl.ds(col, 16)] = ov
        return 0

    lax.fori_loop(0, CHUNK * (CW // 16), ostep, 0)
    plsc.subcore_barrier()

    pltpu.sync_copy(dsts.at[c].at[s], idx_d)

    def step(j, _):
        pltpu.async_copy(ones_b, acc.at[idx_d.at[j]], sem_s, add=True).wait()
        return 0

    lax.fori_loop(0, CPW2, step, 0)
    plsc.subcore_barrier()

    pltpu.sync_copy(acc.at[pl.ds(s * RPW, RPW)],
                    out.at[c].at[pl.ds(s * RPW, RPW)])


def _sc_count(dsts):
    kern = pl.kernel(
        _sc_count_body,
        out_type=jax.ShapeDtypeStruct((NC, NPAD, CW), jnp.float32),
        mesh=plsc.VectorSubcoreMesh(core_axis_name="c", subcore_axis_name="s",
                                    num_cores=NC, num_subcores=NS),
        scratch_types=[
            pltpu.VMEM((CPW2, CHUNK), jnp.int32),
            pltpu.VMEM((CHUNK, CW), jnp.float32),
            pltpu.VMEM_SHARED((NPAD, CW), jnp.float32),
            pltpu.SemaphoreType.DMA,
        ],
    )
    return kern(dsts)


def _sc_agg(table, src, dst):
    kern = pl.kernel(
        _sc_agg_body,
        out_type=jax.ShapeDtypeStruct((NC, NPAD, D), jnp.float32),
        mesh=plsc.VectorSubcoreMesh(core_axis_name="c", subcore_axis_name="s",
                                    num_cores=NC, num_subcores=NS),
        scratch_types=[
            pltpu.VMEM((CPW, CHUNK), jnp.int32),
            pltpu.VMEM((CPW, CHUNK), jnp.int32),
            pltpu.VMEM((NBUF, CHUNK, D), jnp.float32),
            pltpu.VMEM_SHARED((NPAD, D), jnp.float32),
            pltpu.SemaphoreType.DMA((NBUF,)),
            pltpu.SemaphoreType.DMA((NBUF,)),
        ],
    )
    return kern(table, src, dst)


_PREC = lax.Precision.HIGHEST


def _mm(x, w):
    return lax.dot_general(x, w, (((1,), (1,)), ((), ())),
                           precision=_PREC, preferred_element_type=jnp.float32)


def _prologue_body(xn_ref, xr_ref, wl0_ref, wl1_ref, y2_ref):
    y2_ref[0] = _mm(xn_ref[...], wl0_ref[...])
    y2_ref[1] = _mm(xr_ref[...], wl1_ref[...])


def _prologue(xn, xr, wl0, wl1, blk=1000):
    grid = (N_N // blk,)
    return pl.pallas_call(
        _prologue_body,
        grid=grid,
        in_specs=[
            pl.BlockSpec((blk, D), lambda i: (i, 0)),
            pl.BlockSpec((blk, D), lambda i: (i, 0)),
            pl.BlockSpec((D, D), lambda i: (0, 0)),
            pl.BlockSpec((D, D), lambda i: (0, 0)),
        ],
        out_specs=pl.BlockSpec((2, blk, D), lambda i: (0, i, 0)),
        out_shape=jax.ShapeDtypeStruct((2, N_N, D), jnp.float32),
    )(xn, xr, wl0, wl1)


def _mean_from_partials(sp_ref, cp_ref):
    ssum = sp_ref[0] + sp_ref[1]
    cnt = cp_ref[0][:, 0:1]
    return ssum / jnp.maximum(cnt, 1.0)


def _mean_merged(sp_ref, cp_ref):
    cnt = cp_ref[0][:, 0:1]
    return sp_ref[0] / jnp.maximum(cnt, 1.0)


def _combine_body(spr_ref, cpr_ref, spn_ref, cpn_ref, xr_ref, xn_ref,
                  wr0_ref, wr1_ref, b0_ref, b1_ref, wln0_ref, wln1_ref,
                  xr2_ref, xn2_ref, y2_ref):
    mean_r = _mean_merged(spr_ref, cpr_ref)
    mean_n = _mean_merged(spn_ref, cpn_ref)
    xr2 = mean_r + b0_ref[...] + _mm(xr_ref[...], wr0_ref[...])
    xn2 = mean_n + b1_ref[...] + _mm(xn_ref[...], wr1_ref[...])
    xr2_ref[...] = xr2
    xn2_ref[...] = xn2
    y2_ref[0] = _mm(xn2, wln0_ref[...])
    y2_ref[1] = _mm(xr2, wln1_ref[...])


def _cnt_spec(rel, blk):
    return pl.BlockSpec((1, blk, CW), lambda i, r=rel: (r, i, 0))


def _combine(sp2, cnt, xr, xn, wr0, wr1, b0, b1, wln0, wln1,
             blk=1000):
    grid = (N_R // blk,)
    rel0 = pl.BlockSpec((1, blk, D), lambda i: (0, i, 0))
    rel1 = pl.BlockSpec((1, blk, D), lambda i: (1, i, 0))
    rows = pl.BlockSpec((blk, D), lambda i: (i, 0))
    wspec = pl.BlockSpec((D, D), lambda i: (0, 0))
    bspec = pl.BlockSpec((1, D), lambda i: (0, 0))
    return pl.pallas_call(
        _combine_body,
        grid=grid,
        in_specs=[rel0, _cnt_spec(0, blk), rel1, _cnt_spec(1, blk),
                  rows, rows, wspec, wspec, bspec, bspec, wspec, wspec],
        out_specs=[rows, rows, pl.BlockSpec((2, blk, D), lambda i: (0, i, 0))],
        out_shape=[jax.ShapeDtypeStruct((N_R, D), jnp.float32),
                   jax.ShapeDtypeStruct((N_R, D), jnp.float32),
                   jax.ShapeDtypeStruct((2, N_R, D), jnp.float32)],
    )(sp2, cnt, sp2, cnt, xr, xn, wr0, wr1, b0, b1, wln0, wln1)


def _final_body(spr_ref, cpr_ref, xr_ref, wr0_ref, b0_ref,
                wre_ref, bre_ref, notes_ref, wno_ref, bno_ref, out_ref):
    mean_r = _mean_from_partials(spr_ref, cpr_ref)
    xr2 = mean_r + b0_ref[...] + _mm(xr_ref[...], wr0_ref[...])
    a = _mm(xr2, wre_ref[...]) + bre_ref[...]
    b = _mm(notes_ref[...], wno_ref[...]) + bno_ref[...]
    eps = 1e-8
    num = jnp.sum(a * b, axis=1)
    na = jnp.maximum(jnp.sqrt(jnp.sum(a * a, axis=1)), eps)
    nb = jnp.maximum(jnp.sqrt(jnp.sum(b * b, axis=1)), eps)
    out_ref[...] = ((num / (na * nb) + 1.0) * 0.5).reshape(out_ref.shape)


def _final(spr, cnt, xr, wr0, b0, wre, bre, notes, wno, bno, blk=1000):
    grid = (N_R // blk,)
    part = pl.BlockSpec((NC, blk, D), lambda i: (0, i, 0))
    rows = pl.BlockSpec((blk, D), lambda i: (i, 0))
    wspec = pl.BlockSpec((D, D), lambda i: (0, 0))
    bspec = pl.BlockSpec((1, D), lambda i: (0, 0))
    return pl.pallas_call(
        _final_body,
        grid=grid,
        in_specs=[part, _cnt_spec(0, blk), rows, wspec, bspec,
                  wspec, bspec, rows, wspec, bspec],
        out_specs=pl.BlockSpec((1, 1, blk), lambda i: (i, 0, 0)),
        out_shape=jax.ShapeDtypeStruct((N_R // blk, 1, blk), jnp.float32),
    )(spr, cnt, xr, wr0, b0, wre, bre, notes, wno, bno)


def _prep_edges(edge_index):
    src = edge_index[0].astype(jnp.int32)
    dst = edge_index[1].astype(jnp.int32)
    pad = EPAD - E
    src = jnp.concatenate([src, jnp.zeros((pad,), jnp.int32)])
    dst = jnp.concatenate([dst, jnp.full((pad,), PADV, jnp.int32)])
    return src.reshape(NW, CPW, CHUNK), dst.reshape(NW, CPW, CHUNK)


def kernel(x_reaction, x_node, output_notes_opt, edge_index_n2r,
           edge_index_r2n, Wl, bl, Wr, W_reaction, b_reaction, W_nodes,
           b_nodes):
    src_nr, dst_nr = _prep_edges(edge_index_n2r)
    src_rn, dst_rn = _prep_edges(edge_index_r2n)

    dsts = jnp.stack([dst_nr.reshape(NS, CPW2, CHUNK),
                      dst_rn.reshape(NS, CPW2, CHUNK)])
    cnt = _sc_count(dsts)

    idxp = jnp.stack(
        [(src_nr + (dst_nr << 16)).reshape(NS, CPW2, CHUNK),
         (src_rn + (dst_rn << 16)).reshape(NS, CPW2, CHUNK)])

    xr, xn = x_reaction, x_node
    y2 = _prologue(xn, xr, Wl[0, 0], Wl[0, 1])
    for i in range(L - 1):
        sp2 = _sc_agg2(y2.reshape(2 * N_N, D), idxp)
        xr, xn, y2 = _combine(
            sp2, cnt, xr, xn,
            Wr[i, 0], Wr[i, 1],
            bl[i, 0].reshape(1, D), bl[i, 1].reshape(1, D),
            Wl[i + 1, 0], Wl[i + 1, 1])
    spr = _sc_agg(y2.reshape(2 * N_N, D), src_nr, dst_nr)
    out = _final(spr, cnt, xr, Wr[L - 1, 0], bl[L - 1, 0].reshape(1, D),
                 W_reaction, b_reaction.reshape(1, D),
                 output_notes_opt, W_nodes, b_nodes.reshape(1, D))
    return out.reshape(N_R)
